# Initial kernel scaffold; baseline (speedup 1.0000x reference)
import jax
import jax.numpy as jnp
from jax.experimental import pallas as pl


def kernel(m_data, W_val, b_val, feature_embed, universe_embed, intervention_flag):
    raise NotImplementedError("write your pallas kernel here")


# speedup vs baseline: 19.9034x; 19.9034x over previous
import functools

import jax
import jax.numpy as jnp
from jax import lax
from jax.experimental import pallas as pl
from jax.experimental.pallas import tpu as pltpu

U, S, F, D = 27, 1024, 26, 64
RB = 26 * 256
NB = (S * F) // RB


def _body(m_ref, w_ref, b_ref, fe_ref, ue_ref, fl_ref, out_ref):
    u = pl.program_id(0)
    u_ge1 = u >= 1

    fe = fe_ref[...]
    ue_row = jnp.where(u_ge1, ue_ref[1:2, :], ue_ref[0:1, :])
    fidx = lax.broadcasted_iota(jnp.int32, (F, 1), 0)
    mask = (fidx == (u - 1)) & u_ge1
    fl_rows = jnp.where(mask, fl_ref[1:2, :], fl_ref[0:1, :])
    base = fe + b_ref[...][None, :] + ue_row + fl_rows
    base_tile = jnp.broadcast_to(base[None], (RB // F, F, D)).reshape(RB, D)

    m_v = m_ref[0, 0, :]
    w_row = w_ref[0, :]
    out_ref[0] = m_v[:, None] * w_row[None, :] + base_tile


@jax.jit
def kernel(m_data, W_val, b_val, feature_embed, universe_embed, intervention_flag):
    m3 = m_data.reshape(U * NB, 1, RB)
    grid = (U, NB)
    out = pl.pallas_call(
        _body,
        grid=grid,
        in_specs=[
            pl.BlockSpec((1, 1, RB), lambda u, nb: (u * NB + nb, 0, 0)),
            pl.BlockSpec((1, D), lambda u, nb: (0, 0)),
            pl.BlockSpec((D,), lambda u, nb: (0,)),
            pl.BlockSpec((F, D), lambda u, nb: (0, 0)),
            pl.BlockSpec((2, D), lambda u, nb: (0, 0)),
            pl.BlockSpec((2, D), lambda u, nb: (0, 0)),
        ],
        out_specs=pl.BlockSpec((1, RB, D), lambda u, nb: (u, nb, 0)),
        out_shape=jax.ShapeDtypeStruct((U, S * F, D), jnp.float32),
        compiler_params=pltpu.CompilerParams(
            dimension_semantics=("parallel", "parallel"),
        ),
    )(m3, W_val, b_val, feature_embed, universe_embed, intervention_flag)
    return out
